# baseline (speedup 1.0000x reference)
import functools

import jax
import jax.numpy as jnp
from jax import lax
from jax.experimental import pallas as pl
from jax.experimental.pallas import tpu as pltpu
from jax.experimental.pallas import tpu_sc as plsc

T = 16384
S = 16384
D = 128
DR = 144
KNN = 9
LVL = 4
TB = 256
RG = 8
LW = 256
SLAB = 512
NQ = S // SLAB
NSUB = S // LW
GW = LVL * LW
KP = 16
INVL2 = 100.0
EPS = 1e-5
BIGF = 3.0e38


def _bf16_rne(x):
    b = jax.lax.bitcast_convert_type(x, jnp.uint32)
    b = b + jnp.uint32(0x7FFF) + ((b >> 16) & jnp.uint32(1))
    b = b & jnp.uint32(0xFFFF0000)
    return jax.lax.bitcast_convert_type(b, jnp.float32)


def _knn_tc_body(tgt_ref, srcT_ref, idx_ref, w_ref, sprep_ref, gv_ref,
                 ge_ref):
    nrg = TB // RG

    @pl.when(pl.program_id(0) == 0)
    def _():
        sx = srcT_ref[0:1, :]
        sy = srcT_ref[1:2, :]
        sz = srcT_ref[2:3, :]
        sprep_ref[0:1, :] = _bf16_rne(sx)
        sprep_ref[1:2, :] = _bf16_rne(sy)
        sprep_ref[2:3, :] = _bf16_rne(sz)
        sprep_ref[3:4, :] = (sx * sx + sy * sy) + sz * sz

    def rg_loop(r, _):
        tx = tgt_ref[pl.ds(r * RG, RG), 0:1]
        ty = tgt_ref[pl.ds(r * RG, RG), 1:2]
        tz = tgt_ref[pl.ds(r * RG, RG), 2:3]
        t2 = (tx * tx + ty * ty) + tz * tz
        txb = _bf16_rne(tx)
        tyb = _bf16_rne(ty)
        tzb = _bf16_rne(tz)

        init = tuple([jnp.full((RG, LW), BIGF)] * (2 * LVL))

        def q_loop(qq, carry):
            rv = list(carry[:LVL])
            re = list(carry[LVL:])
            sbase = qq * SLAB
            sx = srcT_ref[0:1, pl.ds(sbase, SLAB)]
            sy = srcT_ref[1:2, pl.ds(sbase, SLAB)]
            sz = srcT_ref[2:3, pl.ds(sbase, SLAB)]
            sxb = sprep_ref[0:1, pl.ds(sbase, SLAB)]
            syb = sprep_ref[1:2, pl.ds(sbase, SLAB)]
            szb = sprep_ref[2:3, pl.ds(sbase, SLAB)]
            s2 = sprep_ref[3:4, pl.ds(sbase, SLAB)]
            dotb = (txb * sxb + tyb * syb) + tzb * szb
            d2m = (t2 + s2) - 2.0 * dotb
            ax = tx - sx
            ay = ty - sy
            az = tz - sz
            d2e = (ax * ax + ay * ay) + az * az
            kb = jax.lax.bitcast_convert_type(d2m, jnp.uint32)
            kb = kb & jnp.uint32(0xFFFFFFC0)
            for cs in range(SLAB // LW):
                sub = qq * (SLAB // LW) + cs
                kbu = lax.slice_in_dim(kb, cs * LW, (cs + 1) * LW, axis=1)
                x = jax.lax.bitcast_convert_type(
                    kbu | jnp.uint32(sub), jnp.float32)
                xe = lax.slice_in_dim(d2e, cs * LW, (cs + 1) * LW, axis=1)
                for l in range(LVL):
                    lt = x < rv[l]
                    nv = jnp.where(lt, x, rv[l])
                    ne = jnp.where(lt, xe, re[l])
                    x = jnp.where(lt, rv[l], x)
                    xe = jnp.where(lt, re[l], xe)
                    rv[l] = nv
                    re[l] = ne
            return tuple(rv) + tuple(re)

        fin = lax.fori_loop(0, NQ, q_loop, init)
        for l in range(LVL):
            gv_ref[pl.ds(r * RG, RG), pl.ds(l * LW, LW)] = fin[l]
            ge_ref[pl.ds(r * RG, RG), pl.ds(l * LW, LW)] = fin[LVL + l]
        return 0

    lax.fori_loop(0, nrg, rg_loop, 0)

    g = gv_ref[...]
    lane = lax.broadcasted_iota(jnp.int32, (TB, GW), 1)
    lane = lax.rem(lane, jnp.int32(LW))
    sub = jax.lax.bitcast_convert_type(g, jnp.uint32) & jnp.uint32(0x3F)
    li = sub.astype(jnp.int32) * LW + lane
    ge = ge_ref[...]

    idxs = []
    vals = []
    for _ in range(KNN):
        m = jnp.min(g, axis=1, keepdims=True)
        cand = jnp.where(g == m, li, jnp.int32(2 * S))
        mi = jnp.min(cand, axis=1, keepdims=True)
        ev = jnp.min(jnp.where(cand == mi, ge, BIGF), axis=1,
                     keepdims=True)
        idxs.append(mi)
        vals.append(ev)
        g = jnp.where(cand == mi, BIGF, g)

    imat = jnp.concatenate(idxs, axis=1)
    vmat = jnp.concatenate(vals, axis=1)
    w = jnp.exp(-INVL2 * vmat)
    w = jnp.where(w < EPS, 0.0, w)
    wsum = jnp.sum(w, axis=1, keepdims=True) + EPS
    w = w / wsum
    idx_ref[...] = jnp.concatenate(
        [imat, jnp.zeros((TB, KP - KNN), jnp.int32)], axis=1)
    w_ref[...] = jnp.concatenate(
        [w, jnp.zeros((TB, KP - KNN), jnp.float32)], axis=1)


def _knn_tc(target_position, srcT):
    return pl.pallas_call(
        _knn_tc_body,
        grid=(T // TB,),
        in_specs=[
            pl.BlockSpec((TB, 3), lambda i: (i, 0)),
            pl.BlockSpec((3, S), lambda i: (0, 0)),
        ],
        out_specs=[
            pl.BlockSpec((TB, KP), lambda i: (i, 0)),
            pl.BlockSpec((TB, KP), lambda i: (i, 0)),
        ],
        out_shape=[
            jax.ShapeDtypeStruct((T, KP), jnp.int32),
            jax.ShapeDtypeStruct((T, KP), jnp.float32),
        ],
        scratch_shapes=[
            pltpu.VMEM((4, S), jnp.float32),
            pltpu.VMEM((TB, GW), jnp.float32),
            pltpu.VMEM((TB, GW), jnp.float32),
        ],
    )(target_position, srcT)


NW = 32
TPW = T // NW
CH = 8
NCHK = TPW // CH
IDXC = CH * KP


NSG = 8
SGW = IDXC // NSG


@functools.lru_cache(maxsize=1)
def _sc_gather_build():
    @functools.partial(
        pl.kernel,
        mesh=plsc.VectorSubcoreMesh(core_axis_name="c", subcore_axis_name="s"),
        out_type=jax.ShapeDtypeStruct((T, D), jnp.float32),
        scratch_types=[
            pltpu.VMEM((TPW * KP,), jnp.int32),
            pltpu.VMEM((TPW, KP), jnp.float32),
            pltpu.VMEM((IDXC, D), jnp.float32),
            pltpu.VMEM((IDXC, D), jnp.float32),
            pltpu.VMEM((CH, D), jnp.float32),
            pltpu.VMEM((CH, D), jnp.float32),
            pltpu.SemaphoreType.DMA,
            pltpu.SemaphoreType.DMA,
            pltpu.SemaphoreType.DMA,
        ],
    )
    def _sc_gather(idx_hbm, w_hbm, tbl_hbm, out_hbm,
                   idx_all, w_all, rows_a, rows_b, out_a, out_b,
                   sem_a, sem_b, sem_o):
        wid = lax.axis_index("s") * 2 + lax.axis_index("c")
        base_t = wid * TPW

        pltpu.sync_copy(idx_hbm.at[pl.ds(base_t * KP, TPW * KP)], idx_all)
        pltpu.sync_copy(w_hbm.at[pl.ds(base_t, TPW)], w_all)

        rows = (rows_a, rows_b)
        outs = (out_a, out_b)
        sems = (sem_a, sem_b)

        iota16 = lax.broadcasted_iota(jnp.int32, (16,), 0)

        def splat(vec, j):
            return vec.at[iota16 * 0 + j].get(mode="promise_in_bounds")

        def fire(c, par):
            for k in range(NSG):
                pltpu.async_copy(
                    tbl_hbm.at[idx_all.at[pl.ds(c * IDXC + k * SGW, SGW)]],
                    rows[par].at[pl.ds(k * SGW, SGW)],
                    sems[par])

        def drain(c, par):
            pltpu.make_async_copy(
                tbl_hbm.at[idx_all.at[pl.ds(c * IDXC, IDXC)]],
                rows[par], sems[par]).wait()

        def compute(c, par):
            rows_v = rows[par]
            ochunk = outs[par]
            wchunk = w_all.at[pl.ds(c * CH, CH)]
            for t in range(CH):
                wrow = wchunk[t, :]
                wjs = [splat(wrow, j) for j in range(KNN)]
                for gseg in range(D // 16):
                    acc = jnp.zeros((16,), jnp.float32)
                    for j in range(KNN):
                        seg = rows_v[t * KP + j, pl.ds(gseg * 16, 16)]
                        acc = acc + wjs[j] * seg
                    ochunk[t, pl.ds(gseg * 16, 16)] = acc
            pltpu.async_copy(
                ochunk, out_hbm.at[pl.ds(base_t + c * CH, CH)], sem_o)

        fire(0, 0)

        def pair(ip, _):
            c0 = ip * 2
            fire(c0 + 1, 1)
            drain(c0, 0)
            compute(c0, 0)
            fire(jnp.minimum(c0 + 2, NCHK - 1), 0)
            drain(c0 + 1, 1)
            compute(c0 + 1, 1)
            pltpu.make_async_copy(
                out_a, out_hbm.at[pl.ds(base_t, CH)], sem_o).wait()
            pltpu.make_async_copy(
                out_b, out_hbm.at[pl.ds(base_t, CH)], sem_o).wait()
            return 0

        lax.fori_loop(0, NCHK // 2, pair, 0)
        pltpu.make_async_copy(
            tbl_hbm.at[idx_all.at[pl.ds(0, IDXC)]], rows_a, sem_a).wait()

    return _sc_gather


def kernel(source_position, target_position, source_values):
    srcT = source_position.T
    idx, w = _knn_tc(target_position, srcT)
    out = _sc_gather_build()(idx.reshape(-1), w, source_values)
    return out


# speedup vs baseline: 1.6810x; 1.6810x over previous
import functools

import jax
import jax.numpy as jnp
from jax import lax
from jax.experimental import pallas as pl
from jax.experimental.pallas import tpu as pltpu
from jax.experimental.pallas import tpu_sc as plsc

T = 16384
S = 16384
D = 128
DR = 144
KNN = 9
LVL = 4
TB = 256
RG = 8
LW = 256
SLAB = 512
NQ = S // SLAB
NSUB = S // LW
GW = LVL * LW
KP = 16
INVL2 = 100.0
EPS = 1e-5
BIGF = 3.0e38


def _bf16_rne(x):
    b = jax.lax.bitcast_convert_type(x, jnp.uint32)
    b = b + jnp.uint32(0x7FFF) + ((b >> 16) & jnp.uint32(1))
    b = b & jnp.uint32(0xFFFF0000)
    return jax.lax.bitcast_convert_type(b, jnp.float32)


def _knn_tc_body(tgt_ref, srcT_ref, idx_ref, w_ref, sprep_ref, gv_ref,
                 ge_ref):
    nrg = TB // RG

    @pl.when(pl.program_id(0) == 0)
    def _():
        sx = srcT_ref[0:1, :]
        sy = srcT_ref[1:2, :]
        sz = srcT_ref[2:3, :]
        sprep_ref[0:1, :] = _bf16_rne(sx)
        sprep_ref[1:2, :] = _bf16_rne(sy)
        sprep_ref[2:3, :] = _bf16_rne(sz)
        sprep_ref[3:4, :] = (sx * sx + sy * sy) + sz * sz

    def rg_loop(r, _):
        tx = tgt_ref[pl.ds(r * RG, RG), 0:1]
        ty = tgt_ref[pl.ds(r * RG, RG), 1:2]
        tz = tgt_ref[pl.ds(r * RG, RG), 2:3]
        t2 = (tx * tx + ty * ty) + tz * tz
        txb = _bf16_rne(tx)
        tyb = _bf16_rne(ty)
        tzb = _bf16_rne(tz)

        init = tuple([jnp.full((RG, LW), BIGF)] * (2 * LVL))

        def q_loop(qq, carry):
            rv = list(carry[:LVL])
            re = list(carry[LVL:])
            sbase = qq * SLAB
            sx = srcT_ref[0:1, pl.ds(sbase, SLAB)]
            sy = srcT_ref[1:2, pl.ds(sbase, SLAB)]
            sz = srcT_ref[2:3, pl.ds(sbase, SLAB)]
            sxb = sprep_ref[0:1, pl.ds(sbase, SLAB)]
            syb = sprep_ref[1:2, pl.ds(sbase, SLAB)]
            szb = sprep_ref[2:3, pl.ds(sbase, SLAB)]
            s2 = sprep_ref[3:4, pl.ds(sbase, SLAB)]
            dotb = (txb * sxb + tyb * syb) + tzb * szb
            d2m = (t2 + s2) - 2.0 * dotb
            ax = tx - sx
            ay = ty - sy
            az = tz - sz
            d2e = (ax * ax + ay * ay) + az * az
            kb = jax.lax.bitcast_convert_type(d2m, jnp.uint32)
            kb = kb & jnp.uint32(0xFFFFFFC0)
            for cs in range(SLAB // LW):
                sub = qq * (SLAB // LW) + cs
                kbu = lax.slice_in_dim(kb, cs * LW, (cs + 1) * LW, axis=1)
                x = jax.lax.bitcast_convert_type(
                    kbu | jnp.uint32(sub), jnp.float32)
                xe = lax.slice_in_dim(d2e, cs * LW, (cs + 1) * LW, axis=1)
                for l in range(LVL):
                    lt = x < rv[l]
                    nv = jnp.where(lt, x, rv[l])
                    ne = jnp.where(lt, xe, re[l])
                    x = jnp.where(lt, rv[l], x)
                    xe = jnp.where(lt, re[l], xe)
                    rv[l] = nv
                    re[l] = ne
            return tuple(rv) + tuple(re)

        fin = lax.fori_loop(0, NQ, q_loop, init)
        for l in range(LVL):
            gv_ref[pl.ds(r * RG, RG), pl.ds(l * LW, LW)] = fin[l]
            ge_ref[pl.ds(r * RG, RG), pl.ds(l * LW, LW)] = fin[LVL + l]
        return 0

    lax.fori_loop(0, nrg, rg_loop, 0)

    g = gv_ref[...]
    lane = lax.broadcasted_iota(jnp.int32, (TB, GW), 1)
    lane = lax.rem(lane, jnp.int32(LW))
    sub = jax.lax.bitcast_convert_type(g, jnp.uint32) & jnp.uint32(0x3F)
    li = sub.astype(jnp.int32) * LW + lane
    ge = ge_ref[...]

    idxs = []
    vals = []
    for _ in range(KNN):
        m = jnp.min(g, axis=1, keepdims=True)
        cand = jnp.where(g == m, li, jnp.int32(2 * S))
        mi = jnp.min(cand, axis=1, keepdims=True)
        ev = jnp.min(jnp.where(cand == mi, ge, BIGF), axis=1,
                     keepdims=True)
        idxs.append(mi)
        vals.append(ev)
        g = jnp.where(cand == mi, BIGF, g)

    imat = jnp.concatenate(idxs, axis=1)
    vmat = jnp.concatenate(vals, axis=1)
    w = jnp.exp(-INVL2 * vmat)
    w = jnp.where(w < EPS, 0.0, w)
    wsum = jnp.sum(w, axis=1, keepdims=True) + EPS
    w = w / wsum
    idx_ref[...] = imat
    w_ref[...] = jnp.concatenate(
        [w, jnp.zeros((TB, KP - KNN), jnp.float32)], axis=1)


def _knn_tc(target_position, srcT):
    return pl.pallas_call(
        _knn_tc_body,
        grid=(T // TB,),
        in_specs=[
            pl.BlockSpec((TB, 3), lambda i: (i, 0)),
            pl.BlockSpec((3, S), lambda i: (0, 0)),
        ],
        out_specs=[
            pl.BlockSpec((TB, KNN), lambda i: (i, 0)),
            pl.BlockSpec((TB, KP), lambda i: (i, 0)),
        ],
        out_shape=[
            jax.ShapeDtypeStruct((T, KNN), jnp.int32),
            jax.ShapeDtypeStruct((T, KP), jnp.float32),
        ],
        scratch_shapes=[
            pltpu.VMEM((4, S), jnp.float32),
            pltpu.VMEM((TB, GW), jnp.float32),
            pltpu.VMEM((TB, GW), jnp.float32),
        ],
    )(target_position, srcT)


NW = 32
TPW = T // NW
CH = 8
NCHK = TPW // CH
IDXC = CH * KNN


NSG = 9
SGW = IDXC // NSG


@functools.lru_cache(maxsize=1)
def _sc_gather_build():
    @functools.partial(
        pl.kernel,
        mesh=plsc.VectorSubcoreMesh(core_axis_name="c", subcore_axis_name="s"),
        out_type=jax.ShapeDtypeStruct((T, D), jnp.float32),
        scratch_types=[
            pltpu.VMEM((TPW * KNN,), jnp.int32),
            pltpu.VMEM((TPW, KP), jnp.float32),
            pltpu.VMEM((IDXC, D), jnp.float32),
            pltpu.VMEM((IDXC, D), jnp.float32),
            pltpu.VMEM((CH, D), jnp.float32),
            pltpu.VMEM((CH, D), jnp.float32),
            pltpu.SemaphoreType.DMA,
            pltpu.SemaphoreType.DMA,
            pltpu.SemaphoreType.DMA,
        ],
    )
    def _sc_gather(idx_hbm, w_hbm, tbl_hbm, out_hbm,
                   idx_all, w_all, rows_a, rows_b, out_a, out_b,
                   sem_a, sem_b, sem_o):
        wid = lax.axis_index("s") * 2 + lax.axis_index("c")
        base_t = wid * TPW

        pltpu.sync_copy(idx_hbm.at[pl.ds(base_t * KNN, TPW * KNN)], idx_all)
        pltpu.sync_copy(w_hbm.at[pl.ds(base_t, TPW)], w_all)

        rows = (rows_a, rows_b)
        outs = (out_a, out_b)
        sems = (sem_a, sem_b)

        iota16 = lax.broadcasted_iota(jnp.int32, (16,), 0)

        def splat(vec, j):
            return vec.at[iota16 * 0 + j].get(mode="promise_in_bounds")

        def fire(c, par):
            for k in range(NSG):
                pltpu.async_copy(
                    tbl_hbm.at[idx_all.at[pl.ds(c * IDXC + k * SGW, SGW)]],
                    rows[par].at[pl.ds(k * SGW, SGW)],
                    sems[par])

        def drain(c, par):
            pltpu.make_async_copy(
                tbl_hbm.at[idx_all.at[pl.ds(c * IDXC, IDXC)]],
                rows[par], sems[par]).wait()

        def compute(c, par):
            rows_v = rows[par]
            ochunk = outs[par]
            wchunk = w_all.at[pl.ds(c * CH, CH)]
            for t in range(CH):
                wrow = wchunk[t, :]
                wjs = [splat(wrow, j) for j in range(KNN)]
                for gseg in range(D // 16):
                    acc = jnp.zeros((16,), jnp.float32)
                    for j in range(KNN):
                        seg = rows_v[t * KNN + j, pl.ds(gseg * 16, 16)]
                        acc = acc + wjs[j] * seg
                    ochunk[t, pl.ds(gseg * 16, 16)] = acc
            pltpu.async_copy(
                ochunk, out_hbm.at[pl.ds(base_t + c * CH, CH)], sem_o)

        fire(0, 0)

        def pair(ip, _):
            c0 = ip * 2
            fire(c0 + 1, 1)
            drain(c0, 0)
            compute(c0, 0)
            fire(jnp.minimum(c0 + 2, NCHK - 1), 0)
            drain(c0 + 1, 1)
            compute(c0 + 1, 1)
            pltpu.make_async_copy(
                out_a, out_hbm.at[pl.ds(base_t, CH)], sem_o).wait()
            pltpu.make_async_copy(
                out_b, out_hbm.at[pl.ds(base_t, CH)], sem_o).wait()
            return 0

        lax.fori_loop(0, NCHK // 2, pair, 0)
        pltpu.make_async_copy(
            tbl_hbm.at[idx_all.at[pl.ds(0, IDXC)]], rows_a, sem_a).wait()

    return _sc_gather


def kernel(source_position, target_position, source_values):
    srcT = source_position.T
    idx, w = _knn_tc(target_position, srcT)
    out = _sc_gather_build()(idx.reshape(-1), w, source_values)
    return out
